# baseline (speedup 1.0000x reference)
import functools

import jax
import jax.numpy as jnp
from jax import lax
from jax.experimental import pallas as pl
from jax.experimental.pallas import tpu as pltpu
from jax.experimental.pallas import tpu_sc as plsc

_TP = 2
_NG = 2
_NS = 2


@functools.lru_cache(maxsize=None)
def _build(hist, batch, dim):
    info = plsc.get_sparse_core_info()
    nc, ns, nl = info.num_cores, info.num_subcores, info.num_lanes
    nw = nc * ns
    nstep = hist // _TP
    assert batch == nw * 128 and dim % 8 == 0 and nstep % _NG == 0
    nblk = batch // 128
    ndg = dim // 8

    mesh = plsc.VectorSubcoreMesh(core_axis_name="c", subcore_axis_name="s")

    @functools.partial(
        pl.kernel,
        out_type=jax.ShapeDtypeStruct((hist, ndg, nblk, 8, 128), jnp.float32),
        mesh=mesh,
        scratch_types=[
            pltpu.VMEM((hist, 128), jnp.int32),
            pltpu.VMEM((_NG, _TP, 128, dim), jnp.float32),
            pltpu.VMEM((_NS, _TP, ndg, 8, 128), jnp.float32),
            [pltpu.SemaphoreType.DMA] * _NG,
            [pltpu.SemaphoreType.DMA] * _NS,
        ],
        compiler_params=pltpu.CompilerParams(
            use_tc_tiling_on_sc=False, needs_layout_passes=False),
    )
    def gather_kernel(idxt_hbm, table_hbm, out_hbm, idx_v, rows_v, tbuf_v,
                      gsem, ssem):
        w = lax.axis_index("s") * nc + lax.axis_index("c")
        pltpu.sync_copy(idxt_hbm.at[:, pl.ds(w * 128, 128)], idx_v)

        lane = lax.iota(jnp.int32, nl)
        ridx = [lane + j * nl for j in range(128 // nl)]

        def start_gather(m, g):
            for tt in range(_TP):
                pltpu.async_copy(table_hbm.at[idx_v.at[m * _TP + tt]],
                                 rows_v.at[g, tt], gsem[g])

        def wait_gather(g):
            for tt in range(_TP):
                pltpu.make_async_copy(
                    table_hbm.at[idx_v.at[0]], rows_v.at[g, tt],
                    gsem[g]).wait()

        def start_store(m, s):
            pltpu.async_copy(tbuf_v.at[s],
                             out_hbm.at[pl.ds(m * _TP, _TP), :, w], ssem[s])

        def wait_store(s):
            pltpu.make_async_copy(
                tbuf_v.at[s], out_hbm.at[pl.ds(0, _TP), :, w], ssem[s]).wait()

        def transpose(g, s):
            rows = rows_v.at[g]
            tbuf = tbuf_v.at[s]

            @plsc.parallel_loop(0, _TP * ndg, unroll=2)
            def trans_q(q):
                tt = lax.div(q, ndg)
                dg = lax.rem(q, ndg)
                tsel = jnp.broadcast_to(tt, (nl,))
                for ds in range(8):
                    cidx = jnp.broadcast_to(dg * 8 + ds, (nl,))
                    for j in range(128 // nl):
                        v = plsc.load_gather(rows, [tsel, ridx[j], cidx])
                        tbuf[tt, dg, ds, pl.ds(j * nl, nl)] = v

        for g in range(_NG - 1):
            start_gather(g, g)

        def body(u, carry):
            m0 = u * _NG
            for r in range(_NG):
                m = m0 + r
                g, s = r % _NG, r % _NS
                wait_gather(g)

                @pl.when(m + (_NG - 1) < nstep)
                def _():
                    start_gather(m + (_NG - 1), (g + _NG - 1) % _NG)

                @pl.when(m >= _NS)
                def _():
                    wait_store(s)

                transpose(g, s)
                start_store(m, s)
            return carry

        lax.fori_loop(0, nstep // _NG, body, 0)
        for s in range(_NS):
            wait_store(s)

    return gather_kernel


def kernel(indices, table):
    batch, hist = indices.shape
    dim = table.shape[1]
    idx_t = indices.T.astype(jnp.int32)
    tmp = _build(hist, batch, dim)(idx_t, table)
    return tmp.transpose(2, 4, 0, 1, 3).reshape(batch, hist, dim)


# speedup vs baseline: 1.8016x; 1.8016x over previous
import functools

import jax
import jax.numpy as jnp
from jax import lax
from jax.experimental import pallas as pl
from jax.experimental.pallas import tpu as pltpu
from jax.experimental.pallas import tpu_sc as plsc

_TP = 2
_NG = 2
_NS = 2


@functools.lru_cache(maxsize=None)
def _build(hist, batch, dim):
    info = plsc.get_sparse_core_info()
    nc, ns, nl = info.num_cores, info.num_subcores, info.num_lanes
    nw = nc * ns
    nstep = hist // _TP
    assert batch == nw * 128 and dim % 8 == 0 and nstep % _NG == 0
    nblk = batch // 128
    ndg = dim // 8

    mesh = plsc.VectorSubcoreMesh(core_axis_name="c", subcore_axis_name="s")

    @functools.partial(
        pl.kernel,
        out_type=jax.ShapeDtypeStruct((hist, ndg, nblk, 8, 128), jnp.float32),
        mesh=mesh,
        scratch_types=[
            pltpu.VMEM((hist, 128), jnp.int32),
            pltpu.VMEM((_NG, _TP, 128, dim), jnp.float32),
            pltpu.VMEM((_NS, _TP, ndg, 8, 129), jnp.float32),
            [pltpu.SemaphoreType.DMA] * _NG,
            [pltpu.SemaphoreType.DMA] * _NS,
        ],
        compiler_params=pltpu.CompilerParams(
            use_tc_tiling_on_sc=False, needs_layout_passes=False),
    )
    def gather_kernel(idxt_hbm, table_hbm, out_hbm, idx_v, rows_v, tbuf_v,
                      gsem, ssem):
        w = lax.axis_index("s") * nc + lax.axis_index("c")
        pltpu.sync_copy(idxt_hbm.at[:, pl.ds(w * 128, 128)], idx_v)

        lane = lax.iota(jnp.int32, nl)
        ridx = [lane + j * nl for j in range(128 // nl)]

        def start_gather(m, g):
            for tt in range(_TP):
                pltpu.async_copy(table_hbm.at[idx_v.at[m * _TP + tt]],
                                 rows_v.at[g, tt], gsem[g])

        def wait_gather(g):
            for tt in range(_TP):
                pltpu.make_async_copy(
                    table_hbm.at[idx_v.at[0]], rows_v.at[g, tt],
                    gsem[g]).wait()

        def start_store(m, s):
            pltpu.async_copy(tbuf_v.at[s, :, :, :, pl.ds(0, 128)],
                             out_hbm.at[pl.ds(m * _TP, _TP), :, w], ssem[s])

        def wait_store(s):
            pltpu.make_async_copy(
                tbuf_v.at[s, :, :, :, pl.ds(0, 128)],
                out_hbm.at[pl.ds(0, _TP), :, w], ssem[s]).wait()

        dgi = [(lane + k * nl) >> 3 for k in range(dim // nl)]
        dsi = [(lane + k * nl) & 7 for k in range(dim // nl)]
        tsel = [jnp.broadcast_to(jnp.int32(tt), (nl,)) for tt in range(_TP)]

        def transpose(g, s):
            tbuf = tbuf_v.at[s]

            @plsc.parallel_loop(0, 128, unroll=2)
            def trans_r(r):
                bidx = jnp.broadcast_to(r, (nl,))
                for tt in range(_TP):
                    rows = rows_v.at[g, tt]
                    for k in range(dim // nl):
                        v = rows[r, pl.ds(k * nl, nl)]
                        plsc.store_scatter(
                            tbuf, [tsel[tt], dgi[k], dsi[k], bidx], v)

        for g in range(_NG - 1):
            start_gather(g, g)

        def body(u, carry):
            m0 = u * _NG
            for r in range(_NG):
                m = m0 + r
                g, s = r % _NG, r % _NS
                wait_gather(g)

                @pl.when(m + (_NG - 1) < nstep)
                def _():
                    start_gather(m + (_NG - 1), (g + _NG - 1) % _NG)

                @pl.when(m >= _NS)
                def _():
                    wait_store(s)

                transpose(g, s)
                start_store(m, s)
            return carry

        lax.fori_loop(0, nstep // _NG, body, 0)
        for s in range(_NS):
            wait_store(s)

    return gather_kernel


def kernel(indices, table):
    batch, hist = indices.shape
    dim = table.shape[1]
    idx_t = indices.T.astype(jnp.int32)
    tmp = _build(hist, batch, dim)(idx_t, table)
    return tmp.transpose(2, 4, 0, 1, 3).reshape(batch, hist, dim)
